# baseline (speedup 1.0000x reference)
import functools

import jax
import jax.numpy as jnp
from jax import lax
from jax.experimental import pallas as pl
from jax.experimental.pallas import tpu as pltpu
from jax.experimental.pallas import tpu_sc as plsc

N = 10000
E = 160000
D = 256
B = 10
NPG = N // B
K = NPG // 2


def _rank_body(agg_ref, deg0_ref, deg1_ref, b_ref, rank_ref, g_ref):
    deg = deg0_ref[0] + deg1_ref[0]
    norm = jnp.where(deg > 0, lax.rsqrt(jnp.maximum(deg, 1.0)), 0.0)
    s = agg_ref[0] * norm + b_ref[0, 0]
    st = jnp.transpose(s, (1, 0))
    gt = st > s
    eq = st == s
    ii = lax.broadcasted_iota(jnp.int32, (NPG, NPG), 0)
    jj = lax.broadcasted_iota(jnp.int32, (NPG, NPG), 1)
    mask = gt | (eq & (ii < jj))
    rank_ref[0] = jnp.sum(mask.astype(jnp.int32), axis=0, keepdims=True)
    g_ref[0] = jnp.tanh(s)


def _ranks_tc(agg2d, deg_p0, deg_p1, b2d):
    blk = pl.BlockSpec((1, 1, NPG), lambda g: (g, 0, 0))
    rank3, g3 = pl.pallas_call(
        _rank_body,
        grid=(B,),
        in_specs=[blk, blk, blk, pl.BlockSpec((1, 1), lambda g: (0, 0))],
        out_specs=[blk, blk],
        out_shape=[
            jax.ShapeDtypeStruct((B, 1, NPG), jnp.int32),
            jax.ShapeDtypeStruct((B, 1, NPG), jnp.float32),
        ],
    )(agg2d.reshape(B, 1, NPG), deg_p0.reshape(B, 1, NPG),
      deg_p1.reshape(B, 1, NPG), b2d)
    return rank3.reshape(B, NPG), g3.reshape(B, NPG)


_ROWS_PER_W = 160
_PERM_PAD = 5120


def _pool_body(rank_hbm, g_hbm, feat_hbm, perm_hbm, pooled_hbm,
               rankv, permv, permslice, posv, gvals, rows, shared_perm, sem):
    nc = 2
    c = lax.axis_index("c")
    s = lax.axis_index("s")
    w = s * nc + c

    @pl.when(s == 0)
    def _build():
        zero16 = jnp.zeros((16,), jnp.int32)
        @pl.loop(0, _PERM_PAD // 16)
        def _z(i):
            permv[pl.ds(i * 16, 16)] = zero16

        pltpu.sync_copy(rank_hbm, rankv)

        @pl.loop(0, N // 16)
        def _scatter(i):
            r16 = rankv[pl.ds(i * 16, 16)]
            nodes = lax.iota(jnp.int32, 16) + i * 16
            graph = lax.shift_right_logical(nodes * 8389, 23)
            pos = graph * K + r16
            plsc.store_scatter(permv, [pos], nodes, mask=r16 < K)

        @pl.when(c == 0)
        def _wp():
            pltpu.sync_copy(permv.at[pl.ds(0, B * K)], perm_hbm)

        pltpu.sync_copy(permv, shared_perm)

    plsc.subcore_barrier()

    base = w * _ROWS_PER_W
    pltpu.sync_copy(shared_perm.at[pl.ds(base, _ROWS_PER_W)], permslice)
    cps = []
    for ch in range(2):
        idx = permslice.at[pl.ds(ch * 80, 80)]
        cps.append(pltpu.async_copy(
            feat_hbm.at[idx], rows.at[pl.ds(ch * 80, 80)], sem))
        cps.append(pltpu.async_copy(g_hbm.at[idx], gvals.at[pl.ds(ch * 80, 80)], sem))
    for cp in cps:
        cp.wait()

    @pl.loop(0, _ROWS_PER_W)
    def _scale(r):
        splat = plsc.load_gather(gvals, [jnp.full((16,), 0, jnp.int32) + r])
        for k in range(D // 16):
            rows[r, pl.ds(k * 16, 16)] = rows[r, pl.ds(k * 16, 16)] * splat

    @pl.when(w < 31)
    def _wfull():
        pltpu.sync_copy(rows.at[pl.ds(0, _ROWS_PER_W)],
                        pooled_hbm.at[pl.ds(base, _ROWS_PER_W)])

    @pl.when(w == 31)
    def _wtail():
        pltpu.sync_copy(rows.at[pl.ds(0, 40)], pooled_hbm.at[pl.ds(4960, 40)])


def _pool_sc(rank_flat, g_flat, feature):
    mesh = plsc.VectorSubcoreMesh(core_axis_name="c", subcore_axis_name="s")
    f = pl.kernel(
        _pool_body,
        out_type=[
            jax.ShapeDtypeStruct((B * K,), jnp.int32),
            jax.ShapeDtypeStruct((B * K, D), jnp.float32),
        ],
        mesh=mesh,
        compiler_params=pltpu.CompilerParams(needs_layout_passes=False),
        scratch_types=[
            pltpu.VMEM((N,), jnp.int32),
            pltpu.VMEM((_PERM_PAD,), jnp.int32),
            pltpu.VMEM((_ROWS_PER_W,), jnp.int32),
            pltpu.VMEM((_ROWS_PER_W,), jnp.int32),
            pltpu.VMEM((_ROWS_PER_W,), jnp.float32),
            pltpu.VMEM((_ROWS_PER_W, D), jnp.float32),
            pltpu.VMEM_SHARED((_PERM_PAD,), jnp.int32),
            pltpu.SemaphoreType.DMA,
        ],
    )
    return f(rank_flat, g_flat, feature)


def kernel(feature, edge_index, W, b):
    src = edge_index[0]
    dst = edge_index[1]
    deg_out = jnp.zeros((N,), jnp.float32).at[src].add(1.0)
    deg_in = jnp.zeros((N,), jnp.float32).at[dst].add(1.0)
    norm_src = jnp.where(deg_out > 0, lax.rsqrt(jnp.maximum(deg_out, 1.0)), 0.0)
    h = feature @ W
    h = h * norm_src[:, None]
    agg = jnp.zeros((N, 1), jnp.float32).at[dst].add(h[src])

    agg2d = agg[:, 0].reshape(B, NPG)
    b2d = b.reshape(1, 1)
    rank, g = _ranks_tc(agg2d, deg_in, jnp.zeros((N,), jnp.float32), b2d)

    perm, pooled = _pool_sc(rank.reshape(-1), g.reshape(-1), feature)
    next_batch_num_nodes = jnp.full((B,), K, dtype=jnp.int32)
    return pooled, perm, next_batch_num_nodes


# speedup vs baseline: 1.1909x; 1.1909x over previous
import functools

import jax
import jax.numpy as jnp
from jax import lax
from jax.experimental import pallas as pl
from jax.experimental.pallas import tpu as pltpu
from jax.experimental.pallas import tpu_sc as plsc

N = 10000
E = 160000
D = 256
B = 10
NPG = N // B
K = NPG // 2


_EPW = E // 32
_STRIPE = 640
_HPAD = 16 * _STRIPE


def _deg_body(ei_hbm, deg_hbm,
              edges, hist, acc, tmp, shared, sem):
    nc = 2
    c = lax.axis_index("c")
    s = lax.axis_index("s")
    w = s * nc + c

    for which in range(2):
        pltpu.sync_copy(ei_hbm.at[pl.ds(which * E + w * _EPW, _EPW)], edges)

        zero16 = jnp.zeros((16,), jnp.float32)

        @pl.loop(0, _HPAD // 16)
        def _z(i):
            hist[pl.ds(i * 16, 16)] = zero16

        @pl.loop(0, _EPW // 16)
        def _count(i):
            idx = edges[pl.ds(i * 16, 16)]
            cnt, last = plsc.scan_count(idx)
            plsc.addupdate_scatter(hist, [idx], cnt.astype(jnp.float32),
                                   mask=last)

        rem = _EPW % 16
        if rem:
            lanes = lax.iota(jnp.int32, 16)
            idx = edges[pl.ds(_EPW - 16, 16)]
            inmask = lanes >= (16 - rem)
            cnt, last = plsc.scan_count(idx, inmask)
            plsc.addupdate_scatter(hist, [idx], cnt.astype(jnp.float32),
                                   mask=last & inmask)

        pltpu.sync_copy(hist, shared.at[pl.ds(s * _HPAD, _HPAD)])
        plsc.subcore_barrier()

        @pl.loop(0, _STRIPE // 16)
        def _z2(i):
            acc[pl.ds(i * 16, 16)] = zero16

        for t in range(16):
            pltpu.sync_copy(
                shared.at[pl.ds(t * _HPAD + s * _STRIPE, _STRIPE)], tmp)

            @pl.loop(0, _STRIPE // 16)
            def _add(i):
                acc[pl.ds(i * 16, 16)] = (
                    acc[pl.ds(i * 16, 16)] + tmp[pl.ds(i * 16, 16)])

        pltpu.sync_copy(
            acc,
            deg_hbm.at[pl.ds((which * 2 + c) * _HPAD + s * _STRIPE, _STRIPE)])
        plsc.subcore_barrier()


def _degrees_sc(edge_index):
    mesh = plsc.VectorSubcoreMesh(core_axis_name="c", subcore_axis_name="s")
    f = pl.kernel(
        _deg_body,
        out_type=jax.ShapeDtypeStruct((4 * _HPAD,), jnp.float32),
        mesh=mesh,
        compiler_params=pltpu.CompilerParams(needs_layout_passes=False),
        scratch_types=[
            pltpu.VMEM((_EPW,), jnp.int32),
            pltpu.VMEM((_HPAD,), jnp.float32),
            pltpu.VMEM((_STRIPE,), jnp.float32),
            pltpu.VMEM((_STRIPE,), jnp.float32),
            pltpu.VMEM_SHARED((16 * _HPAD,), jnp.float32),
            pltpu.SemaphoreType.DMA,
        ],
    )
    deg = f(edge_index.reshape(-1))
    return deg.reshape(4, _HPAD)


def _hs_body(feat_ref, w_ref, d0_ref, d1_ref, hs_ref):
    deg = d0_ref[...] + d1_ref[...]
    norm = jnp.where(deg > 0, lax.rsqrt(jnp.maximum(deg, 1.0)), 0.0)
    h0 = jnp.dot(feat_ref[...].astype(jnp.bfloat16),
                 w_ref[...].astype(jnp.bfloat16),
                 preferred_element_type=jnp.float32)
    hs_ref[...] = h0 * norm


def _hs_tc(feature, W, deg_p0, deg_p1):
    rblk = 1000
    return pl.pallas_call(
        _hs_body,
        grid=(N // rblk,),
        in_specs=[
            pl.BlockSpec((rblk, D), lambda g: (g, 0)),
            pl.BlockSpec((D, 1), lambda g: (0, 0)),
            pl.BlockSpec((rblk, 1), lambda g: (g, 0)),
            pl.BlockSpec((rblk, 1), lambda g: (g, 0)),
        ],
        out_specs=pl.BlockSpec((rblk, 1), lambda g: (g, 0)),
        out_shape=jax.ShapeDtypeStruct((N, 1), jnp.float32),
    )(feature, W, deg_p0.reshape(N, 1), deg_p1.reshape(N, 1))


def _rank_body(agg_ref, deg0_ref, deg1_ref, b_ref, rank_ref, g_ref):
    deg = deg0_ref[0] + deg1_ref[0]
    norm = jnp.where(deg > 0, lax.rsqrt(jnp.maximum(deg, 1.0)), 0.0)
    s = agg_ref[0] * norm + b_ref[0, 0]
    st = jnp.transpose(s, (1, 0))
    gt = st > s
    eq = st == s
    ii = lax.broadcasted_iota(jnp.int32, (NPG, NPG), 0)
    jj = lax.broadcasted_iota(jnp.int32, (NPG, NPG), 1)
    mask = gt | (eq & (ii < jj))
    rank_ref[0] = jnp.sum(mask.astype(jnp.int32), axis=0, keepdims=True)
    g_ref[0] = jnp.tanh(s)


def _ranks_tc(agg2d, deg_p0, deg_p1, b2d):
    blk = pl.BlockSpec((1, 1, NPG), lambda g: (g, 0, 0))
    rank3, g3 = pl.pallas_call(
        _rank_body,
        grid=(B,),
        in_specs=[blk, blk, blk, pl.BlockSpec((1, 1), lambda g: (0, 0))],
        out_specs=[blk, blk],
        out_shape=[
            jax.ShapeDtypeStruct((B, 1, NPG), jnp.int32),
            jax.ShapeDtypeStruct((B, 1, NPG), jnp.float32),
        ],
    )(agg2d.reshape(B, 1, NPG), deg_p0.reshape(B, 1, NPG),
      deg_p1.reshape(B, 1, NPG), b2d)
    return rank3.reshape(B, NPG), g3.reshape(B, NPG)


_ROWS_PER_W = 160
_PERM_PAD = 5120


def _pool_body(rank_hbm, g_hbm, feat_hbm, perm_hbm, pooled_hbm,
               rankv, permv, permslice, posv, gvals, rows, shared_perm, sem):
    nc = 2
    c = lax.axis_index("c")
    s = lax.axis_index("s")
    w = s * nc + c

    @pl.when(s == 0)
    def _build():
        zero16 = jnp.zeros((16,), jnp.int32)
        @pl.loop(0, _PERM_PAD // 16)
        def _z(i):
            permv[pl.ds(i * 16, 16)] = zero16

        pltpu.sync_copy(rank_hbm, rankv)

        @pl.loop(0, N // 16)
        def _scatter(i):
            r16 = rankv[pl.ds(i * 16, 16)]
            nodes = lax.iota(jnp.int32, 16) + i * 16
            graph = lax.shift_right_logical(nodes * 8389, 23)
            pos = graph * K + r16
            plsc.store_scatter(permv, [pos], nodes, mask=r16 < K)

        @pl.when(c == 0)
        def _wp():
            pltpu.sync_copy(permv.at[pl.ds(0, B * K)], perm_hbm)

        pltpu.sync_copy(permv, shared_perm)

    plsc.subcore_barrier()

    base = w * _ROWS_PER_W
    pltpu.sync_copy(shared_perm.at[pl.ds(base, _ROWS_PER_W)], permslice)
    cps = []
    for ch in range(2):
        idx = permslice.at[pl.ds(ch * 80, 80)]
        cps.append(pltpu.async_copy(
            feat_hbm.at[idx], rows.at[pl.ds(ch * 80, 80)], sem))
        cps.append(pltpu.async_copy(g_hbm.at[idx], gvals.at[pl.ds(ch * 80, 80)], sem))
    for cp in cps:
        cp.wait()

    @pl.loop(0, _ROWS_PER_W)
    def _scale(r):
        splat = plsc.load_gather(gvals, [jnp.full((16,), 0, jnp.int32) + r])
        for k in range(D // 16):
            rows[r, pl.ds(k * 16, 16)] = rows[r, pl.ds(k * 16, 16)] * splat

    @pl.when(w < 31)
    def _wfull():
        pltpu.sync_copy(rows.at[pl.ds(0, _ROWS_PER_W)],
                        pooled_hbm.at[pl.ds(base, _ROWS_PER_W)])

    @pl.when(w == 31)
    def _wtail():
        pltpu.sync_copy(rows.at[pl.ds(0, 40)], pooled_hbm.at[pl.ds(4960, 40)])


def _pool_sc(rank_flat, g_flat, feature):
    mesh = plsc.VectorSubcoreMesh(core_axis_name="c", subcore_axis_name="s")
    f = pl.kernel(
        _pool_body,
        out_type=[
            jax.ShapeDtypeStruct((B * K,), jnp.int32),
            jax.ShapeDtypeStruct((B * K, D), jnp.float32),
        ],
        mesh=mesh,
        compiler_params=pltpu.CompilerParams(needs_layout_passes=False),
        scratch_types=[
            pltpu.VMEM((N,), jnp.int32),
            pltpu.VMEM((_PERM_PAD,), jnp.int32),
            pltpu.VMEM((_ROWS_PER_W,), jnp.int32),
            pltpu.VMEM((_ROWS_PER_W,), jnp.int32),
            pltpu.VMEM((_ROWS_PER_W,), jnp.float32),
            pltpu.VMEM((_ROWS_PER_W, D), jnp.float32),
            pltpu.VMEM_SHARED((_PERM_PAD,), jnp.int32),
            pltpu.SemaphoreType.DMA,
        ],
    )
    return f(rank_flat, g_flat, feature)


def kernel(feature, edge_index, W, b):
    src = edge_index[0]
    dst = edge_index[1]
    deg4 = _degrees_sc(edge_index)
    h = _hs_tc(feature, W, deg4[0, :N], deg4[1, :N])
    agg = jnp.zeros((N, 1), jnp.float32).at[dst].add(h[src])

    agg2d = agg[:, 0].reshape(B, NPG)
    b2d = b.reshape(1, 1)
    rank, g = _ranks_tc(agg2d, deg4[2, :N], deg4[3, :N], b2d)

    perm, pooled = _pool_sc(rank.reshape(-1), g.reshape(-1), feature)
    next_batch_num_nodes = jnp.full((B,), K, dtype=jnp.int32)
    return pooled, perm, next_batch_num_nodes


# speedup vs baseline: 1.1949x; 1.0033x over previous
import functools

import jax
import jax.numpy as jnp
from jax import lax
from jax.experimental import pallas as pl
from jax.experimental.pallas import tpu as pltpu
from jax.experimental.pallas import tpu_sc as plsc

N = 10000
E = 160000
D = 256
B = 10
NPG = N // B
K = NPG // 2


_EPW = E // 32
_STRIPE = 640
_HPAD = 16 * _STRIPE


def _deg_body(ei_hbm, deg_hbm,
              edges, hist, acc, tmp, shared, sem):
    nc = 2
    c = lax.axis_index("c")
    s = lax.axis_index("s")
    w = s * nc + c

    for which in range(2):
        pltpu.sync_copy(ei_hbm.at[pl.ds(which * E + w * _EPW, _EPW)], edges)

        zero16 = jnp.zeros((16,), jnp.float32)

        @pl.loop(0, _HPAD // 16)
        def _z(i):
            hist[pl.ds(i * 16, 16)] = zero16

        @pl.loop(0, _EPW // 16)
        def _count(i):
            idx = edges[pl.ds(i * 16, 16)]
            cnt, last = plsc.scan_count(idx)
            plsc.addupdate_scatter(hist, [idx], cnt.astype(jnp.float32),
                                   mask=last)

        rem = _EPW % 16
        if rem:
            lanes = lax.iota(jnp.int32, 16)
            idx = edges[pl.ds(_EPW - 16, 16)]
            inmask = lanes >= (16 - rem)
            cnt, last = plsc.scan_count(idx, inmask)
            plsc.addupdate_scatter(hist, [idx], cnt.astype(jnp.float32),
                                   mask=last & inmask)

        pltpu.sync_copy(hist, shared.at[pl.ds(s * _HPAD, _HPAD)])
        plsc.subcore_barrier()

        @pl.loop(0, _STRIPE // 16)
        def _z2(i):
            acc[pl.ds(i * 16, 16)] = zero16

        for t in range(16):
            pltpu.sync_copy(
                shared.at[pl.ds(t * _HPAD + s * _STRIPE, _STRIPE)], tmp)

            @pl.loop(0, _STRIPE // 16)
            def _add(i):
                acc[pl.ds(i * 16, 16)] = (
                    acc[pl.ds(i * 16, 16)] + tmp[pl.ds(i * 16, 16)])

        pltpu.sync_copy(
            acc,
            deg_hbm.at[pl.ds((which * 2 + c) * _HPAD + s * _STRIPE, _STRIPE)])
        plsc.subcore_barrier()


def _degrees_sc(edge_index):
    mesh = plsc.VectorSubcoreMesh(core_axis_name="c", subcore_axis_name="s")
    f = pl.kernel(
        _deg_body,
        out_type=jax.ShapeDtypeStruct((4 * _HPAD,), jnp.float32),
        mesh=mesh,
        compiler_params=pltpu.CompilerParams(needs_layout_passes=False),
        scratch_types=[
            pltpu.VMEM((_EPW,), jnp.int32),
            pltpu.VMEM((_HPAD,), jnp.float32),
            pltpu.VMEM((_STRIPE,), jnp.float32),
            pltpu.VMEM((_STRIPE,), jnp.float32),
            pltpu.VMEM_SHARED((16 * _HPAD,), jnp.float32),
            pltpu.SemaphoreType.DMA,
        ],
    )
    deg = f(edge_index.reshape(-1))
    return deg.reshape(4, _HPAD)


def _h0_body(feat_ref, w_ref, h0_ref):
    h0_ref[...] = jnp.dot(feat_ref[...].astype(jnp.bfloat16),
                          w_ref[...].astype(jnp.bfloat16),
                          preferred_element_type=jnp.float32)


def _h0_tc(feature, W):
    rblk = 1000
    return pl.pallas_call(
        _h0_body,
        grid=(N // rblk,),
        in_specs=[
            pl.BlockSpec((rblk, D), lambda g: (g, 0)),
            pl.BlockSpec((D, 1), lambda g: (0, 0)),
        ],
        out_specs=pl.BlockSpec((rblk, 1), lambda g: (g, 0)),
        out_shape=jax.ShapeDtypeStruct((N, 1), jnp.float32),
    )(feature, W)


def _scale_body(h0_ref, d0_ref, d1_ref, hs_ref):
    deg = d0_ref[...] + d1_ref[...]
    norm = jnp.where(deg > 0, lax.rsqrt(jnp.maximum(deg, 1.0)), 0.0)
    hs_ref[...] = h0_ref[...] * norm


def _hs_tc(h0, deg_p0, deg_p1):
    return pl.pallas_call(
        _scale_body,
        in_specs=[
            pl.BlockSpec((N, 1), lambda: (0, 0)),
            pl.BlockSpec((N, 1), lambda: (0, 0)),
            pl.BlockSpec((N, 1), lambda: (0, 0)),
        ],
        out_specs=pl.BlockSpec((N, 1), lambda: (0, 0)),
        out_shape=jax.ShapeDtypeStruct((N, 1), jnp.float32),
    )(h0, deg_p0.reshape(N, 1), deg_p1.reshape(N, 1))


def _rank_body(agg_ref, deg0_ref, deg1_ref, b_ref, rank_ref, g_ref):
    deg = deg0_ref[0] + deg1_ref[0]
    norm = jnp.where(deg > 0, lax.rsqrt(jnp.maximum(deg, 1.0)), 0.0)
    s = agg_ref[0] * norm + b_ref[0, 0]
    st = jnp.transpose(s, (1, 0))
    gt = st > s
    eq = st == s
    ii = lax.broadcasted_iota(jnp.int32, (NPG, NPG), 0)
    jj = lax.broadcasted_iota(jnp.int32, (NPG, NPG), 1)
    mask = gt | (eq & (ii < jj))
    rank_ref[0] = jnp.sum(mask.astype(jnp.int32), axis=0, keepdims=True)
    g_ref[0] = jnp.tanh(s)


def _ranks_tc(agg2d, deg_p0, deg_p1, b2d):
    blk = pl.BlockSpec((1, 1, NPG), lambda g: (g, 0, 0))
    rank3, g3 = pl.pallas_call(
        _rank_body,
        grid=(B,),
        in_specs=[blk, blk, blk, pl.BlockSpec((1, 1), lambda g: (0, 0))],
        out_specs=[blk, blk],
        out_shape=[
            jax.ShapeDtypeStruct((B, 1, NPG), jnp.int32),
            jax.ShapeDtypeStruct((B, 1, NPG), jnp.float32),
        ],
    )(agg2d.reshape(B, 1, NPG), deg_p0.reshape(B, 1, NPG),
      deg_p1.reshape(B, 1, NPG), b2d)
    return rank3.reshape(B, NPG), g3.reshape(B, NPG)


_ROWS_PER_W = 160
_PERM_PAD = 5120


def _pool_body(rank_hbm, g_hbm, feat_hbm, perm_hbm, pooled_hbm,
               rankv, permv, permslice, posv, gvals, rows, shared_perm, sem):
    nc = 2
    c = lax.axis_index("c")
    s = lax.axis_index("s")
    w = s * nc + c

    @pl.when(s == 0)
    def _build():
        zero16 = jnp.zeros((16,), jnp.int32)
        @pl.loop(0, _PERM_PAD // 16)
        def _z(i):
            permv[pl.ds(i * 16, 16)] = zero16

        pltpu.sync_copy(rank_hbm, rankv)

        @pl.loop(0, N // 16)
        def _scatter(i):
            r16 = rankv[pl.ds(i * 16, 16)]
            nodes = lax.iota(jnp.int32, 16) + i * 16
            graph = lax.shift_right_logical(nodes * 8389, 23)
            pos = graph * K + r16
            plsc.store_scatter(permv, [pos], nodes, mask=r16 < K)

        @pl.when(c == 0)
        def _wp():
            pltpu.sync_copy(permv.at[pl.ds(0, B * K)], perm_hbm)

        pltpu.sync_copy(permv, shared_perm)

    plsc.subcore_barrier()

    base = w * _ROWS_PER_W
    pltpu.sync_copy(shared_perm.at[pl.ds(base, _ROWS_PER_W)], permslice)
    cps = []
    for ch in range(2):
        idx = permslice.at[pl.ds(ch * 80, 80)]
        cps.append(pltpu.async_copy(
            feat_hbm.at[idx], rows.at[pl.ds(ch * 80, 80)], sem))
        cps.append(pltpu.async_copy(g_hbm.at[idx], gvals.at[pl.ds(ch * 80, 80)], sem))
    for cp in cps:
        cp.wait()

    @pl.loop(0, _ROWS_PER_W)
    def _scale(r):
        splat = plsc.load_gather(gvals, [jnp.full((16,), 0, jnp.int32) + r])
        for k in range(D // 16):
            rows[r, pl.ds(k * 16, 16)] = rows[r, pl.ds(k * 16, 16)] * splat

    @pl.when(w < 31)
    def _wfull():
        pltpu.sync_copy(rows.at[pl.ds(0, _ROWS_PER_W)],
                        pooled_hbm.at[pl.ds(base, _ROWS_PER_W)])

    @pl.when(w == 31)
    def _wtail():
        pltpu.sync_copy(rows.at[pl.ds(0, 40)], pooled_hbm.at[pl.ds(4960, 40)])


def _pool_sc(rank_flat, g_flat, feature):
    mesh = plsc.VectorSubcoreMesh(core_axis_name="c", subcore_axis_name="s")
    f = pl.kernel(
        _pool_body,
        out_type=[
            jax.ShapeDtypeStruct((B * K,), jnp.int32),
            jax.ShapeDtypeStruct((B * K, D), jnp.float32),
        ],
        mesh=mesh,
        compiler_params=pltpu.CompilerParams(needs_layout_passes=False),
        scratch_types=[
            pltpu.VMEM((N,), jnp.int32),
            pltpu.VMEM((_PERM_PAD,), jnp.int32),
            pltpu.VMEM((_ROWS_PER_W,), jnp.int32),
            pltpu.VMEM((_ROWS_PER_W,), jnp.int32),
            pltpu.VMEM((_ROWS_PER_W,), jnp.float32),
            pltpu.VMEM((_ROWS_PER_W, D), jnp.float32),
            pltpu.VMEM_SHARED((_PERM_PAD,), jnp.int32),
            pltpu.SemaphoreType.DMA,
        ],
    )
    return f(rank_flat, g_flat, feature)


def kernel(feature, edge_index, W, b):
    src = edge_index[0]
    dst = edge_index[1]
    deg4 = _degrees_sc(edge_index)
    h0 = _h0_tc(feature, W)
    h = _hs_tc(h0, deg4[0, :N], deg4[1, :N])
    agg = jnp.zeros((N, 1), jnp.float32).at[dst].add(h[src])

    agg2d = agg[:, 0].reshape(B, NPG)
    b2d = b.reshape(1, 1)
    rank, g = _ranks_tc(agg2d, deg4[2, :N], deg4[3, :N], b2d)

    perm, pooled = _pool_sc(rank.reshape(-1), g.reshape(-1), feature)
    next_batch_num_nodes = jnp.full((B,), K, dtype=jnp.int32)
    return pooled, perm, next_batch_num_nodes
